# baseline (speedup 1.0000x reference)
import functools

import jax
import jax.numpy as jnp
from jax.experimental import pallas as pl
from jax.experimental.pallas import tpu as pltpu

_B = 1024
_D = 1024
_N = 16384
_TEMP_INV = 20.0
_LMAX = 20.0
_SUB = 512
_NSUB = 4
_NT = _SUB * _NSUB
_TILES = _N // _NT


def _loss_body(x_ref, t_ref, f_ref, o_ref, xn_ref, s_ref, te_ref):
    i = pl.program_id(0)

    @pl.when(i == 0)
    def _init():
        x = x_ref[...]
        nrm = jnp.maximum(
            jnp.sqrt(jnp.sum(x * x, axis=1, keepdims=True)), 1e-12)
        xn_ref[...] = x / nrm
        s_ref[...] = jnp.zeros((_B, 1), jnp.float32)
        te_ref[...] = jnp.zeros((_B, 1), jnp.float32)

    s_acc = jnp.zeros((_B, 1), jnp.float32)
    te_acc = jnp.zeros((_B, 1), jnp.float32)
    for j in range(_NSUB):
        l = jax.lax.dot_general(
            xn_ref[...], f_ref[j * _SUB:(j + 1) * _SUB, :],
            (((1,), (1,)), ((), ())),
            preferred_element_type=jnp.float32,
            precision=jax.lax.Precision.DEFAULT)
        e = jnp.exp(l * _TEMP_INV - _LMAX)
        s_acc += jnp.sum(e, axis=1, keepdims=True)
        cols = (i * _NT + j * _SUB
                + jax.lax.broadcasted_iota(jnp.int32, (_B, _SUB), 1))
        hit = cols == t_ref[...]
        te_acc += jnp.sum(jnp.where(hit, e, 0.0), axis=1, keepdims=True)
    s_ref[...] += s_acc
    te_ref[...] += te_acc

    @pl.when(i == _TILES - 1)
    def _fin():
        loss = jnp.log(s_ref[...]) - jnp.log(te_ref[...])
        o_ref[...] = jnp.sum(loss, keepdims=True) * (1.0 / _B)


@functools.partial(jax.jit, static_argnames=())
def kernel(inputs, targets, features):
    out = pl.pallas_call(
        _loss_body,
        grid=(_TILES,),
        in_specs=[
            pl.BlockSpec((_B, _D), lambda i: (0, 0)),
            pl.BlockSpec((_B, 1), lambda i: (0, 0)),
            pl.BlockSpec((_NT, _D), lambda i: (i, 0)),
        ],
        out_specs=pl.BlockSpec((1, 1), lambda i: (0, 0)),
        out_shape=jax.ShapeDtypeStruct((1, 1), jnp.float32),
        scratch_shapes=[
            pltpu.VMEM((_B, _D), jnp.float32),
            pltpu.VMEM((_B, 1), jnp.float32),
            pltpu.VMEM((_B, 1), jnp.float32),
        ],
    )(inputs, targets.astype(jnp.int32).reshape(_B, 1), features)
    return out[0, 0]


# speedup vs baseline: 1.0882x; 1.0882x over previous
import functools

import jax
import jax.numpy as jnp
from jax.experimental import pallas as pl
from jax.experimental.pallas import tpu as pltpu

_B = 1024
_D = 1024
_N = 16384
_TEMP_INV = 20.0
_LMAX = 20.0
_SUB = 1024
_NSUB = 2
_NT = _SUB * _NSUB
_TILES = _N // _NT


def _loss_body(x_ref, t_ref, f_ref, o_ref, xn_ref, s_ref, te_ref):
    i = pl.program_id(0)

    @pl.when(i == 0)
    def _init():
        x = x_ref[...]
        nrm = jnp.maximum(
            jnp.sqrt(jnp.sum(x * x, axis=1, keepdims=True)), 1e-12)
        xn_ref[...] = (x / nrm).astype(jnp.bfloat16)
        s_ref[...] = jnp.zeros((_B, 1), jnp.float32)
        te_ref[...] = jnp.zeros((_B, 1), jnp.float32)

    s_acc = jnp.zeros((_B, 1), jnp.float32)
    te_acc = jnp.zeros((_B, 1), jnp.float32)
    for j in range(_NSUB):
        l = jax.lax.dot_general(
            xn_ref[...], f_ref[j * _SUB:(j + 1) * _SUB, :],
            (((1,), (1,)), ((), ())),
            preferred_element_type=jnp.float32,
            precision=jax.lax.Precision.DEFAULT)
        e = jnp.exp(l * _TEMP_INV - _LMAX)
        s_acc += jnp.sum(e, axis=1, keepdims=True)
        cols = (i * _NT + j * _SUB
                + jax.lax.broadcasted_iota(jnp.int32, (_B, _SUB), 1))
        hit = cols == t_ref[...]
        te_acc += jnp.sum(jnp.where(hit, e, 0.0), axis=1, keepdims=True)
    s_ref[...] += s_acc
    te_ref[...] += te_acc

    @pl.when(i == _TILES - 1)
    def _fin():
        loss = jnp.log(s_ref[...]) - jnp.log(te_ref[...])
        o_ref[...] = jnp.sum(loss, keepdims=True) * (1.0 / _B)


@functools.partial(jax.jit, static_argnames=())
def kernel(inputs, targets, features):
    out = pl.pallas_call(
        _loss_body,
        grid=(_TILES,),
        in_specs=[
            pl.BlockSpec((_B, _D), lambda i: (0, 0)),
            pl.BlockSpec((_B, 1), lambda i: (0, 0)),
            pl.BlockSpec((_NT, _D), lambda i: (i, 0)),
        ],
        out_specs=pl.BlockSpec((1, 1), lambda i: (0, 0)),
        out_shape=jax.ShapeDtypeStruct((1, 1), jnp.float32),
        scratch_shapes=[
            pltpu.VMEM((_B, _D), jnp.bfloat16),
            pltpu.VMEM((_B, 1), jnp.float32),
            pltpu.VMEM((_B, 1), jnp.float32),
        ],
    )(inputs, targets.astype(jnp.int32).reshape(_B, 1), features)
    return out[0, 0]
